# baseline (speedup 1.0000x reference)
import functools

import jax
import jax.numpy as jnp
from jax import lax
from jax.experimental import pallas as pl
from jax.experimental.pallas import tpu as pltpu
from jax.experimental.pallas import tpu_sc as plsc

_B = 4096
_F = 13
_V = 1000
_E = 128
_HID = (1024, 512, 128)
_DIN = _F * _E

_NC = 2
_NS = 16
_NW = _NC * _NS
_BPW = _B // _NW
_CH = _BPW

_TWT = _F * _V


def _sc_gather_body(idxf_hbm, idxb_hbm, emb_hbm, wide_hbm,
                    fm_out, w_out,
                    idxf_v, idxb_v, rows0, rows1, wout_v,
                    sem_g, sem_w):
    w = lax.axis_index("s") * _NC + lax.axis_index("c")
    base = w * _CH

    pltpu.sync_copy(idxf_hbm.at[w], idxf_v)
    pltpu.sync_copy(idxb_hbm.at[w], idxb_v)

    rows = (rows0, rows1)

    wide_cps = [pltpu.async_copy(wide_hbm.at[idxb_v.at[c]], wout_v.at[c],
                                 sem_w)
                for c in range(_F)]

    cps = [None, None]
    cps[0] = pltpu.async_copy(emb_hbm.at[idxf_v.at[0]], rows[0], sem_g)
    for f in range(_F):
        if f + 1 < _F:
            cps[(f + 1) % 2] = pltpu.async_copy(
                emb_hbm.at[idxf_v.at[f + 1]], rows[(f + 1) % 2], sem_g)
        cps[f % 2].wait()
        pltpu.sync_copy(rows[f % 2], fm_out.at[f, pl.ds(base, _CH)])

    for cp in wide_cps:
        cp.wait()
    pltpu.sync_copy(wout_v, w_out.at[w])


def _sc_gather(idxf, idxb, emb, wide):
    mesh = plsc.VectorSubcoreMesh(core_axis_name="c", subcore_axis_name="s",
                                  num_cores=_NC, num_subcores=_NS)
    f32 = jnp.float32
    return pl.kernel(
        _sc_gather_body,
        out_type=(
            jax.ShapeDtypeStruct((_F, _B, _E), f32),
            jax.ShapeDtypeStruct((_NW, _F, _CH), f32),
        ),
        mesh=mesh,
        scratch_types=[
            pltpu.VMEM((_F, _CH), jnp.int32),
            pltpu.VMEM((_F, _CH), jnp.int32),
            pltpu.VMEM((_CH, _E), f32),
            pltpu.VMEM((_CH, _E), f32),
            pltpu.VMEM((_F, _CH), f32),
            pltpu.SemaphoreType.DMA,
            pltpu.SemaphoreType.DMA,
        ],
        name="two_tower_sc_gather",
    )(idxf, idxb, emb, wide)


def _tower_from_refs(x3_ref, w_ref, W1, b1, W2, b2, W3, b3):
    f32 = jnp.float32
    x3 = [x3_ref[f] for f in range(_F)]
    s = x3[0]
    ss = s * s
    for f in range(1, _F):
        e = x3[f]
        s = s + e
        ss = ss + e * e
    fm = 0.5 * (s * s - ss)
    x = jnp.concatenate(x3, axis=1)
    h = jnp.dot(x.astype(jnp.bfloat16), W1[:],
                preferred_element_type=f32) + b1[:]
    h = jnp.maximum(h, 0.0)
    h = jnp.dot(h.astype(jnp.bfloat16), W2[:],
                preferred_element_type=f32) + b2[:]
    h = jnp.maximum(h, 0.0)
    d = jnp.dot(h.astype(jnp.bfloat16), W3[:],
                preferred_element_type=f32) + b3[:]
    wsum = jnp.sum(w_ref[:], axis=1, keepdims=True)
    return fm, d, wsum


def _tc_user_body(uw_ref, ux_ref, uW1, ub1, uW2, ub2, uW3, ub3,
                  fm_ref, d_ref, ws_ref):
    fm, d, wsum = _tower_from_refs(ux_ref, uw_ref, uW1, ub1, uW2, ub2,
                                   uW3, ub3)
    fm_ref[:] = fm
    d_ref[:] = d
    ws_ref[:] = wsum


def _tc_final_body(iw_ref, ix_ref, iW1, ib1, iW2, ib2, iW3, ib3,
                   fmu_ref, du_ref, wsu_ref, out_ref):
    fm_i, d_i, wi = _tower_from_refs(ix_ref, iw_ref, iW1, ib1, iW2, ib2,
                                     iW3, ib3)
    logit = (wsu_ref[:] * wi
             + jnp.sum(fmu_ref[:] * fm_i, axis=1, keepdims=True)
             + jnp.sum(du_ref[:] * d_i, axis=1, keepdims=True))
    out_ref[:] = jax.nn.sigmoid(logit)


_BS = 512


def _xmap(i):
    return (i, 0)


def _x3map(i):
    return (0, i, 0)


def _wmap(i):
    return (0, 0)


def _tc_user(u_w, u_x, uW1, ub1, uW2, ub2, uW3, ub3):
    f32 = jnp.float32
    full = lambda a: pl.BlockSpec(a.shape, _wmap)
    return pl.pallas_call(
        _tc_user_body,
        grid=(_B // _BS,),
        in_specs=[
            pl.BlockSpec((_BS, _F), _xmap),
            pl.BlockSpec((_F, _BS, _E), _x3map),
        ] + [full(a) for a in (uW1, ub1, uW2, ub2, uW3, ub3)],
        out_specs=(
            pl.BlockSpec((_BS, _E), _xmap),
            pl.BlockSpec((_BS, _E), _xmap),
            pl.BlockSpec((_BS, 1), _xmap),
        ),
        out_shape=(
            jax.ShapeDtypeStruct((_B, _E), f32),
            jax.ShapeDtypeStruct((_B, _E), f32),
            jax.ShapeDtypeStruct((_B, 1), f32),
        ),
        compiler_params=pltpu.CompilerParams(
            dimension_semantics=("arbitrary",),
        ),
        name="two_tower_tc_user",
    )(u_w, u_x, uW1, ub1, uW2, ub2, uW3, ub3)


def _tc_final(i_w, i_x, iW1, ib1, iW2, ib2, iW3, ib3, fm_u, d_u, ws_u):
    f32 = jnp.float32
    full = lambda a: pl.BlockSpec(a.shape, _wmap)
    return pl.pallas_call(
        _tc_final_body,
        grid=(_B // _BS,),
        in_specs=[
            pl.BlockSpec((_BS, _F), _xmap),
            pl.BlockSpec((_F, _BS, _E), _x3map),
        ] + [full(a) for a in (iW1, ib1, iW2, ib2, iW3, ib3)] + [
            pl.BlockSpec((_BS, _E), _xmap),
            pl.BlockSpec((_BS, _E), _xmap),
            pl.BlockSpec((_BS, 1), _xmap),
        ],
        out_specs=pl.BlockSpec((_BS, 1), _xmap),
        out_shape=jax.ShapeDtypeStruct((_B, 1), f32),
        compiler_params=pltpu.CompilerParams(
            dimension_semantics=("arbitrary",),
        ),
        name="two_tower_tc_final",
    )(i_w, i_x, iW1, ib1, iW2, ib2, iW3, ib3, fm_u, d_u, ws_u)


def kernel(inputs, user_emb, user_wide, item_emb, item_wide,
           uW1, ub1, uW2, ub2, uW3, ub3,
           iW1, ib1, iW2, ib2, iW3, ib3):
    i32 = jnp.int32
    bf16 = jnp.bfloat16
    off = (jnp.arange(_F, dtype=i32) * _V)[None, :]
    u_flat = inputs[:, :_F].astype(i32) + off
    i_flat = inputs[:, _F:].astype(i32) + off
    u_idxb = u_flat.reshape(_NW, _F, _CH)
    i_idxb = i_flat.reshape(_NW, _F, _CH)
    u_idxf = u_flat.reshape(_NW, _CH, _F).transpose(0, 2, 1)
    i_idxf = i_flat.reshape(_NW, _CH, _F).transpose(0, 2, 1)

    u_fm, u_wv = _sc_gather(u_idxf, u_idxb,
                            user_emb.reshape(_TWT, _E),
                            user_wide.reshape(_TWT))
    i_fm, i_wv = _sc_gather(i_idxf, i_idxb,
                            item_emb.reshape(_TWT, _E),
                            item_wide.reshape(_TWT))

    fm_u, d_u, ws_u = _tc_user(
        u_wv.reshape(_B, _F), u_fm,
        uW1.astype(bf16), ub1.reshape(1, -1), uW2.astype(bf16),
        ub2.reshape(1, -1), uW3.astype(bf16), ub3.reshape(1, -1))

    pred = _tc_final(
        i_wv.reshape(_B, _F), i_fm,
        iW1.astype(bf16), ib1.reshape(1, -1), iW2.astype(bf16),
        ib2.reshape(1, -1), iW3.astype(bf16), ib3.reshape(1, -1),
        fm_u, d_u, ws_u)
    return pred


# speedup vs baseline: 1.1210x; 1.1210x over previous
import functools

import jax
import jax.numpy as jnp
from jax import lax
from jax.experimental import pallas as pl
from jax.experimental.pallas import tpu as pltpu
from jax.experimental.pallas import tpu_sc as plsc

_B = 4096
_F = 13
_V = 1000
_E = 128
_HID = (1024, 512, 128)
_DIN = _F * _E

_NC = 2
_NS = 16
_NW = _NC * _NS
_BPW = _B // _NW
_CH = _BPW

_TWT = _F * _V


def _sc_gather_body(idxf_hbm, idxb_hbm, emb_hbm, wide_hbm,
                    fm_out, w_out,
                    idxf_v, idxb_v, rows0, rows1, wout_v,
                    sem_g, sem_w):
    w = lax.axis_index("s") * _NC + lax.axis_index("c")
    base = w * _CH

    pltpu.sync_copy(idxf_hbm.at[w], idxf_v)
    pltpu.sync_copy(idxb_hbm.at[w], idxb_v)

    rows = (rows0, rows1)

    wide_cps = [pltpu.async_copy(wide_hbm.at[idxb_v.at[c]], wout_v.at[c],
                                 sem_w)
                for c in range(_F)]

    cps = [None, None]
    cps[0] = pltpu.async_copy(emb_hbm.at[idxf_v.at[0]], rows[0], sem_g)
    for f in range(_F):
        if f + 1 < _F:
            cps[(f + 1) % 2] = pltpu.async_copy(
                emb_hbm.at[idxf_v.at[f + 1]], rows[(f + 1) % 2], sem_g)
        cps[f % 2].wait()
        pltpu.sync_copy(rows[f % 2], fm_out.at[f, pl.ds(base, _CH)])

    for cp in wide_cps:
        cp.wait()
    pltpu.sync_copy(wout_v, w_out.at[w])


def _sc_gather(idxf, idxb, emb, wide):
    mesh = plsc.VectorSubcoreMesh(core_axis_name="c", subcore_axis_name="s",
                                  num_cores=_NC, num_subcores=_NS)
    f32 = jnp.float32
    return pl.kernel(
        _sc_gather_body,
        out_type=(
            jax.ShapeDtypeStruct((_F, _B, _E), f32),
            jax.ShapeDtypeStruct((_NW, _F, _CH), f32),
        ),
        mesh=mesh,
        scratch_types=[
            pltpu.VMEM((_F, _CH), jnp.int32),
            pltpu.VMEM((_F, _CH), jnp.int32),
            pltpu.VMEM((_CH, _E), f32),
            pltpu.VMEM((_CH, _E), f32),
            pltpu.VMEM((_F, _CH), f32),
            pltpu.SemaphoreType.DMA,
            pltpu.SemaphoreType.DMA,
        ],
        name="two_tower_sc_gather",
    )(idxf, idxb, emb, wide)


def _tower_from_refs(x3_ref, w_ref, W1, b1, W2, b2, W3, b3):
    f32 = jnp.float32
    x3 = [x3_ref[f] for f in range(_F)]
    s = x3[0]
    ss = s * s
    for f in range(1, _F):
        e = x3[f]
        s = s + e
        ss = ss + e * e
    fm = 0.5 * (s * s - ss)
    x = jnp.concatenate(x3, axis=1)
    h = jnp.dot(x.astype(jnp.bfloat16), W1[:],
                preferred_element_type=f32) + b1[:]
    h = jnp.maximum(h, 0.0)
    h = jnp.dot(h.astype(jnp.bfloat16), W2[:],
                preferred_element_type=f32) + b2[:]
    h = jnp.maximum(h, 0.0)
    d = jnp.dot(h.astype(jnp.bfloat16), W3[:],
                preferred_element_type=f32) + b3[:]
    wsum = jnp.sum(w_ref[:], axis=1, keepdims=True)
    return fm, d, wsum


def _tc_user_body(uw_ref, ux_ref, uW1, ub1, uW2, ub2, uW3, ub3,
                  fm_ref, d_ref, ws_ref):
    fm, d, wsum = _tower_from_refs(ux_ref, uw_ref, uW1, ub1, uW2, ub2,
                                   uW3, ub3)
    fm_ref[:] = fm
    d_ref[:] = d
    ws_ref[:] = wsum


def _tc_final_body(iw_ref, ix_ref, iW1, ib1, iW2, ib2, iW3, ib3,
                   fmu_ref, du_ref, wsu_ref, out_ref):
    fm_i, d_i, wi = _tower_from_refs(ix_ref, iw_ref, iW1, ib1, iW2, ib2,
                                     iW3, ib3)
    logit = (wsu_ref[:] * wi
             + jnp.sum(fmu_ref[:] * fm_i, axis=1, keepdims=True)
             + jnp.sum(du_ref[:] * d_i, axis=1, keepdims=True))
    out_ref[:] = jax.nn.sigmoid(logit)


_BS = 512


def _xmap(i):
    return (i, 0)


def _x3map(i):
    return (0, i, 0)


def _wmap(i):
    return (0, 0)


def _tc_user(u_w, u_x, uW1, ub1, uW2, ub2, uW3, ub3):
    f32 = jnp.float32
    full = lambda a: pl.BlockSpec(a.shape, _wmap)
    return pl.pallas_call(
        _tc_user_body,
        grid=(_B // _BS,),
        in_specs=[
            pl.BlockSpec((_BS, _F), _xmap),
            pl.BlockSpec((_F, _BS, _E), _x3map),
        ] + [full(a) for a in (uW1, ub1, uW2, ub2, uW3, ub3)],
        out_specs=(
            pl.BlockSpec((_BS, _E), _xmap),
            pl.BlockSpec((_BS, _E), _xmap),
            pl.BlockSpec((_BS, 1), _xmap),
        ),
        out_shape=(
            jax.ShapeDtypeStruct((_B, _E), f32),
            jax.ShapeDtypeStruct((_B, _E), f32),
            jax.ShapeDtypeStruct((_B, 1), f32),
        ),
        compiler_params=pltpu.CompilerParams(
            dimension_semantics=("arbitrary",),
        ),
        name="two_tower_tc_user",
    )(u_w, u_x, uW1, ub1, uW2, ub2, uW3, ub3)


def _tc_final(i_w, i_x, iW1, ib1, iW2, ib2, iW3, ib3, fm_u, d_u, ws_u):
    f32 = jnp.float32
    full = lambda a: pl.BlockSpec(a.shape, _wmap)
    return pl.pallas_call(
        _tc_final_body,
        grid=(_B // _BS,),
        in_specs=[
            pl.BlockSpec((_BS, _F), _xmap),
            pl.BlockSpec((_F, _BS, _E), _x3map),
        ] + [full(a) for a in (iW1, ib1, iW2, ib2, iW3, ib3)] + [
            pl.BlockSpec((_BS, _E), _xmap),
            pl.BlockSpec((_BS, _E), _xmap),
            pl.BlockSpec((_BS, 1), _xmap),
        ],
        out_specs=pl.BlockSpec((_BS, 1), _xmap),
        out_shape=jax.ShapeDtypeStruct((_B, 1), f32),
        compiler_params=pltpu.CompilerParams(
            dimension_semantics=("arbitrary",),
        ),
        name="two_tower_tc_final",
    )(i_w, i_x, iW1, ib1, iW2, ib2, iW3, ib3, fm_u, d_u, ws_u)


def kernel(inputs, user_emb, user_wide, item_emb, item_wide,
           uW1, ub1, uW2, ub2, uW3, ub3,
           iW1, ib1, iW2, ib2, iW3, ib3):
    i32 = jnp.int32
    bf16 = jnp.bfloat16
    off = (jnp.arange(_F, dtype=i32) * _V)[None, :]
    u_flat = inputs[:, :_F].astype(i32) + off
    i_flat = inputs[:, _F:].astype(i32) + off
    u_idxb = u_flat.reshape(_NW, _F, _CH)
    i_idxb = i_flat.reshape(_NW, _F, _CH)
    u_idxf = u_flat.reshape(_NW, _CH, _F).transpose(0, 2, 1)
    i_idxf = i_flat.reshape(_NW, _CH, _F).transpose(0, 2, 1)

    u_fm, u_wv = _sc_gather(u_idxf, u_idxb,
                            user_emb.reshape(_TWT, _E),
                            user_wide.reshape(_TWT))
    i_fm, i_wv = _sc_gather(i_idxf, i_idxb,
                            item_emb.reshape(_TWT, _E),
                            item_wide.reshape(_TWT))

    fm_u, d_u, ws_u = _tc_user(
        u_wv.reshape(_B, _F), u_fm,
        uW1.astype(bf16), ub1.reshape(1, -1), uW2.astype(bf16),
        ub2.reshape(1, -1), uW3.astype(bf16), ub3.reshape(1, -1))

    i_fm, i_wv, fm_u = lax.optimization_barrier((i_fm, i_wv, fm_u))

    pred = _tc_final(
        i_wv.reshape(_B, _F), i_fm,
        iW1.astype(bf16), ib1.reshape(1, -1), iW2.astype(bf16),
        ib2.reshape(1, -1), iW3.astype(bf16), ib3.reshape(1, -1),
        fm_u, d_u, ws_u)
    return pred


# speedup vs baseline: 1.1268x; 1.0052x over previous
import functools

import jax
import jax.numpy as jnp
from jax import lax
from jax.experimental import pallas as pl
from jax.experimental.pallas import tpu as pltpu
from jax.experimental.pallas import tpu_sc as plsc

_B = 4096
_F = 13
_V = 1000
_E = 128
_HID = (1024, 512, 128)
_DIN = _F * _E

_NC = 2
_NS = 16
_NW = _NC * _NS
_BPW = _B // _NW
_CH = _BPW

_TWT = _F * _V


def _sc_gather_body(idxf_hbm, idxb_hbm, emb_hbm, wide_hbm,
                    fm_out, w_out,
                    idxf_v, idxb_v, rows0, rows1, wout_v,
                    sem_g, sem_w):
    w = lax.axis_index("s") * _NC + lax.axis_index("c")
    base = w * _CH

    pltpu.sync_copy(idxf_hbm.at[w], idxf_v)
    pltpu.sync_copy(idxb_hbm.at[w], idxb_v)

    rows = (rows0, rows1)

    wide_cps = [pltpu.async_copy(wide_hbm.at[idxb_v.at[c]], wout_v.at[c],
                                 sem_w)
                for c in range(_F)]

    cps = [None, None]
    cps[0] = pltpu.async_copy(emb_hbm.at[idxf_v.at[0]], rows[0], sem_g)
    for f in range(_F):
        if f + 1 < _F:
            cps[(f + 1) % 2] = pltpu.async_copy(
                emb_hbm.at[idxf_v.at[f + 1]], rows[(f + 1) % 2], sem_g)
        cps[f % 2].wait()
        pltpu.sync_copy(rows[f % 2], fm_out.at[f, pl.ds(base, _CH)])

    for cp in wide_cps:
        cp.wait()
    pltpu.sync_copy(wout_v, w_out.at[w])


def _sc_gather(idxf, idxb, emb, wide):
    mesh = plsc.VectorSubcoreMesh(core_axis_name="c", subcore_axis_name="s",
                                  num_cores=_NC, num_subcores=_NS)
    f32 = jnp.float32
    return pl.kernel(
        _sc_gather_body,
        out_type=(
            jax.ShapeDtypeStruct((_F, _B, _E), f32),
            jax.ShapeDtypeStruct((_NW, _F, _CH), f32),
        ),
        mesh=mesh,
        scratch_types=[
            pltpu.VMEM((_F, _CH), jnp.int32),
            pltpu.VMEM((_F, _CH), jnp.int32),
            pltpu.VMEM((_CH, _E), f32),
            pltpu.VMEM((_CH, _E), f32),
            pltpu.VMEM((_F, _CH), f32),
            pltpu.SemaphoreType.DMA,
            pltpu.SemaphoreType.DMA,
        ],
        name="two_tower_sc_gather",
    )(idxf, idxb, emb, wide)


def _tower_from_refs(x3_ref, w_ref, W1, b1, W2, b2, W3, b3):
    f32 = jnp.float32
    x3 = [x3_ref[f] for f in range(_F)]
    s = x3[0]
    ss = s * s
    for f in range(1, _F):
        e = x3[f]
        s = s + e
        ss = ss + e * e
    fm = 0.5 * (s * s - ss)
    x = jnp.concatenate(x3, axis=1)
    h = jnp.dot(x.astype(jnp.bfloat16), W1[:],
                preferred_element_type=f32) + b1[:]
    h = jnp.maximum(h, 0.0)
    h = jnp.dot(h.astype(jnp.bfloat16), W2[:],
                preferred_element_type=f32) + b2[:]
    h = jnp.maximum(h, 0.0)
    d = jnp.dot(h.astype(jnp.bfloat16), W3[:],
                preferred_element_type=f32) + b3[:]
    wsum = jnp.sum(w_ref[:], axis=1, keepdims=True)
    return fm, d, wsum


def _tc_user_body(uw_ref, ux_ref, uW1, ub1, uW2, ub2, uW3, ub3,
                  fm_ref, d_ref, ws_ref):
    fm, d, wsum = _tower_from_refs(ux_ref, uw_ref, uW1, ub1, uW2, ub2,
                                   uW3, ub3)
    fm_ref[:] = fm
    d_ref[:] = d
    ws_ref[:] = wsum


def _tc_final_body(iw_ref, ix_ref, iW1, ib1, iW2, ib2, iW3, ib3,
                   fmu_ref, du_ref, wsu_ref, out_ref):
    fm_i, d_i, wi = _tower_from_refs(ix_ref, iw_ref, iW1, ib1, iW2, ib2,
                                     iW3, ib3)
    logit = (wsu_ref[:] * wi
             + jnp.sum(fmu_ref[:] * fm_i, axis=1, keepdims=True)
             + jnp.sum(du_ref[:] * d_i, axis=1, keepdims=True))
    out_ref[:] = jax.nn.sigmoid(logit)


_BS = 1024


def _xmap(i):
    return (i, 0)


def _x3map(i):
    return (0, i, 0)


def _wmap(i):
    return (0, 0)


def _tc_user(u_w, u_x, uW1, ub1, uW2, ub2, uW3, ub3):
    f32 = jnp.float32
    full = lambda a: pl.BlockSpec(a.shape, _wmap)
    return pl.pallas_call(
        _tc_user_body,
        grid=(_B // _BS,),
        in_specs=[
            pl.BlockSpec((_BS, _F), _xmap),
            pl.BlockSpec((_F, _BS, _E), _x3map),
        ] + [full(a) for a in (uW1, ub1, uW2, ub2, uW3, ub3)],
        out_specs=(
            pl.BlockSpec((_BS, _E), _xmap),
            pl.BlockSpec((_BS, _E), _xmap),
            pl.BlockSpec((_BS, 1), _xmap),
        ),
        out_shape=(
            jax.ShapeDtypeStruct((_B, _E), f32),
            jax.ShapeDtypeStruct((_B, _E), f32),
            jax.ShapeDtypeStruct((_B, 1), f32),
        ),
        compiler_params=pltpu.CompilerParams(
            dimension_semantics=("arbitrary",),
        ),
        name="two_tower_tc_user",
    )(u_w, u_x, uW1, ub1, uW2, ub2, uW3, ub3)


def _tc_final(i_w, i_x, iW1, ib1, iW2, ib2, iW3, ib3, fm_u, d_u, ws_u):
    f32 = jnp.float32
    full = lambda a: pl.BlockSpec(a.shape, _wmap)
    return pl.pallas_call(
        _tc_final_body,
        grid=(_B // _BS,),
        in_specs=[
            pl.BlockSpec((_BS, _F), _xmap),
            pl.BlockSpec((_F, _BS, _E), _x3map),
        ] + [full(a) for a in (iW1, ib1, iW2, ib2, iW3, ib3)] + [
            pl.BlockSpec((_BS, _E), _xmap),
            pl.BlockSpec((_BS, _E), _xmap),
            pl.BlockSpec((_BS, 1), _xmap),
        ],
        out_specs=pl.BlockSpec((_BS, 1), _xmap),
        out_shape=jax.ShapeDtypeStruct((_B, 1), f32),
        compiler_params=pltpu.CompilerParams(
            dimension_semantics=("arbitrary",),
        ),
        name="two_tower_tc_final",
    )(i_w, i_x, iW1, ib1, iW2, ib2, iW3, ib3, fm_u, d_u, ws_u)


def kernel(inputs, user_emb, user_wide, item_emb, item_wide,
           uW1, ub1, uW2, ub2, uW3, ub3,
           iW1, ib1, iW2, ib2, iW3, ib3):
    i32 = jnp.int32
    bf16 = jnp.bfloat16
    off = (jnp.arange(_F, dtype=i32) * _V)[None, :]
    u_flat = inputs[:, :_F].astype(i32) + off
    i_flat = inputs[:, _F:].astype(i32) + off
    u_idxb = u_flat.reshape(_NW, _F, _CH)
    i_idxb = i_flat.reshape(_NW, _F, _CH)
    u_idxf = u_flat.reshape(_NW, _CH, _F).transpose(0, 2, 1)
    i_idxf = i_flat.reshape(_NW, _CH, _F).transpose(0, 2, 1)

    u_fm, u_wv = _sc_gather(u_idxf, u_idxb,
                            user_emb.reshape(_TWT, _E),
                            user_wide.reshape(_TWT))
    i_fm, i_wv = _sc_gather(i_idxf, i_idxb,
                            item_emb.reshape(_TWT, _E),
                            item_wide.reshape(_TWT))

    fm_u, d_u, ws_u = _tc_user(
        u_wv.reshape(_B, _F), u_fm,
        uW1.astype(bf16), ub1.reshape(1, -1), uW2.astype(bf16),
        ub2.reshape(1, -1), uW3.astype(bf16), ub3.reshape(1, -1))

    i_fm, i_wv, fm_u = lax.optimization_barrier((i_fm, i_wv, fm_u))

    pred = _tc_final(
        i_wv.reshape(_B, _F), i_fm,
        iW1.astype(bf16), ib1.reshape(1, -1), iW2.astype(bf16),
        ib2.reshape(1, -1), iW3.astype(bf16), ib3.reshape(1, -1),
        fm_u, d_u, ws_u)
    return pred


# speedup vs baseline: 1.1666x; 1.0353x over previous
import functools

import jax
import jax.numpy as jnp
from jax import lax
from jax.experimental import pallas as pl
from jax.experimental.pallas import tpu as pltpu
from jax.experimental.pallas import tpu_sc as plsc

_B = 4096
_F = 13
_V = 1000
_E = 128
_HID = (1024, 512, 128)
_DIN = _F * _E

_NC = 2
_NS = 16
_NW = _NC * _NS
_BPW = _B // _NW
_CH = _BPW
_L = 16

_TWT = _F * _V


def _sc_gather_body(idxf_hbm, emb_hbm, wide_hbm,
                    fm_out, ws_out,
                    idxf_v, rows0, rows1, wout_v, wsum_v,
                    sem_g, sem_w):
    w = lax.axis_index("s") * _NC + lax.axis_index("c")
    base = w * _CH

    pltpu.sync_copy(idxf_hbm.at[w], idxf_v)

    rows = (rows0, rows1)

    wide_cps = [pltpu.async_copy(wide_hbm.at[idxf_v.at[f]], wout_v.at[f],
                                 sem_w)
                for f in range(_F)]

    cps = [None, None]
    cps[0] = pltpu.async_copy(emb_hbm.at[idxf_v.at[0]], rows[0], sem_g)
    for f in range(_F):
        if f + 1 < _F:
            cps[(f + 1) % 2] = pltpu.async_copy(
                emb_hbm.at[idxf_v.at[f + 1]], rows[(f + 1) % 2], sem_g)
        cps[f % 2].wait()
        pltpu.sync_copy(rows[f % 2], fm_out.at[f, pl.ds(base, _CH)])

    for cp in wide_cps:
        cp.wait()
    for g in range(_CH // _L):
        acc = wout_v[0, pl.ds(g * _L, _L)]
        for f in range(1, _F):
            acc = acc + wout_v[f, pl.ds(g * _L, _L)]
        wsum_v[pl.ds(g * _L, _L)] = acc
    pltpu.sync_copy(wsum_v, ws_out.at[0, pl.ds(base, _CH)])


def _sc_gather(idxf, emb, wide):
    mesh = plsc.VectorSubcoreMesh(core_axis_name="c", subcore_axis_name="s",
                                  num_cores=_NC, num_subcores=_NS)
    f32 = jnp.float32
    return pl.kernel(
        _sc_gather_body,
        out_type=(
            jax.ShapeDtypeStruct((_F, _B, _E), f32),
            jax.ShapeDtypeStruct((1, _B), f32),
        ),
        mesh=mesh,
        scratch_types=[
            pltpu.VMEM((_F, _CH), jnp.int32),
            pltpu.VMEM((_CH, _E), f32),
            pltpu.VMEM((_CH, _E), f32),
            pltpu.VMEM((_F, _CH), f32),
            pltpu.VMEM((_CH,), f32),
            pltpu.SemaphoreType.DMA,
            pltpu.SemaphoreType.DMA,
        ],
        name="two_tower_sc_gather",
    )(idxf, emb, wide)


def _tower_from_refs(x3_ref, W1, b1, W2, b2, W3, b3):
    f32 = jnp.float32
    x3 = [x3_ref[f] for f in range(_F)]
    s = x3[0]
    ss = s * s
    for f in range(1, _F):
        e = x3[f]
        s = s + e
        ss = ss + e * e
    fm = 0.5 * (s * s - ss)
    x = jnp.concatenate(x3, axis=1)
    h = jnp.dot(x.astype(jnp.bfloat16), W1[:],
                preferred_element_type=f32) + b1[:]
    h = jnp.maximum(h, 0.0)
    h = jnp.dot(h.astype(jnp.bfloat16), W2[:],
                preferred_element_type=f32) + b2[:]
    h = jnp.maximum(h, 0.0)
    d = jnp.dot(h.astype(jnp.bfloat16), W3[:],
                preferred_element_type=f32) + b3[:]
    return fm, d


def _tc_user_body(ux_ref, uW1, ub1, uW2, ub2, uW3, ub3,
                  fm_ref, d_ref):
    fm, d = _tower_from_refs(ux_ref, uW1, ub1, uW2, ub2, uW3, ub3)
    fm_ref[:] = fm
    d_ref[:] = d


def _tc_final_body(ix_ref, iW1, ib1, iW2, ib2, iW3, ib3,
                   fmu_ref, du_ref, wsu_ref, wsi_ref, out_ref):
    fm_i, d_i = _tower_from_refs(ix_ref, iW1, ib1, iW2, ib2, iW3, ib3)
    logit = (wsu_ref[:] * wsi_ref[:]
             + jnp.sum(fmu_ref[:] * fm_i, axis=1, keepdims=True)
             + jnp.sum(du_ref[:] * d_i, axis=1, keepdims=True))
    out_ref[:] = jax.nn.sigmoid(logit)


_BS = 1024


def _xmap(i):
    return (i, 0)


def _x3map(i):
    return (0, i, 0)


def _wmap(i):
    return (0, 0)


def _tc_user(u_x, uW1, ub1, uW2, ub2, uW3, ub3):
    f32 = jnp.float32
    full = lambda a: pl.BlockSpec(a.shape, _wmap)
    return pl.pallas_call(
        _tc_user_body,
        grid=(_B // _BS,),
        in_specs=[
            pl.BlockSpec((_F, _BS, _E), _x3map),
        ] + [full(a) for a in (uW1, ub1, uW2, ub2, uW3, ub3)],
        out_specs=(
            pl.BlockSpec((_BS, _E), _xmap),
            pl.BlockSpec((_BS, _E), _xmap),
        ),
        out_shape=(
            jax.ShapeDtypeStruct((_B, _E), f32),
            jax.ShapeDtypeStruct((_B, _E), f32),
        ),
        compiler_params=pltpu.CompilerParams(
            dimension_semantics=("arbitrary",),
        ),
        name="two_tower_tc_user",
    )(u_x, uW1, ub1, uW2, ub2, uW3, ub3)


def _tc_final(i_x, iW1, ib1, iW2, ib2, iW3, ib3, fm_u, d_u, ws_u, ws_i):
    f32 = jnp.float32
    full = lambda a: pl.BlockSpec(a.shape, _wmap)
    return pl.pallas_call(
        _tc_final_body,
        grid=(_B // _BS,),
        in_specs=[
            pl.BlockSpec((_F, _BS, _E), _x3map),
        ] + [full(a) for a in (iW1, ib1, iW2, ib2, iW3, ib3)] + [
            pl.BlockSpec((_BS, _E), _xmap),
            pl.BlockSpec((_BS, _E), _xmap),
            pl.BlockSpec((_BS, 1), _xmap),
            pl.BlockSpec((_BS, 1), _xmap),
        ],
        out_specs=pl.BlockSpec((_BS, 1), _xmap),
        out_shape=jax.ShapeDtypeStruct((_B, 1), f32),
        compiler_params=pltpu.CompilerParams(
            dimension_semantics=("arbitrary",),
        ),
        name="two_tower_tc_final",
    )(i_x, iW1, ib1, iW2, ib2, iW3, ib3, fm_u, d_u, ws_u, ws_i)


def kernel(inputs, user_emb, user_wide, item_emb, item_wide,
           uW1, ub1, uW2, ub2, uW3, ub3,
           iW1, ib1, iW2, ib2, iW3, ib3):
    i32 = jnp.int32
    bf16 = jnp.bfloat16
    off = (jnp.arange(_F, dtype=i32) * _V)[None, :]
    u_flat = inputs[:, :_F].astype(i32) + off
    i_flat = inputs[:, _F:].astype(i32) + off
    u_idxf = u_flat.reshape(_NW, _CH, _F).transpose(0, 2, 1)
    i_idxf = i_flat.reshape(_NW, _CH, _F).transpose(0, 2, 1)

    u_fm, u_ws = _sc_gather(u_idxf, user_emb.reshape(_TWT, _E),
                            user_wide.reshape(_TWT))
    i_fm, i_ws = _sc_gather(i_idxf, item_emb.reshape(_TWT, _E),
                            item_wide.reshape(_TWT))

    fm_u, d_u = _tc_user(
        u_fm,
        uW1.astype(bf16), ub1.reshape(1, -1), uW2.astype(bf16),
        ub2.reshape(1, -1), uW3.astype(bf16), ub3.reshape(1, -1))

    i_fm, i_ws, fm_u = lax.optimization_barrier((i_fm, i_ws, fm_u))

    pred = _tc_final(
        i_fm,
        iW1.astype(bf16), ib1.reshape(1, -1), iW2.astype(bf16),
        ib2.reshape(1, -1), iW3.astype(bf16), ib3.reshape(1, -1),
        fm_u, d_u, u_ws.reshape(_B, 1), i_ws.reshape(_B, 1))
    return pred


# speedup vs baseline: 1.1744x; 1.0067x over previous
import functools

import jax
import jax.numpy as jnp
from jax import lax
from jax.experimental import pallas as pl
from jax.experimental.pallas import tpu as pltpu
from jax.experimental.pallas import tpu_sc as plsc

_B = 4096
_F = 13
_V = 1000
_E = 128
_HID = (1024, 512, 128)
_DIN = _F * _E

_NC = 2
_NS = 16
_NW = _NC * _NS
_BPW = _B // _NW
_CH = _BPW
_L = 16

_TWT = _F * _V


def _sc_gather_body(idxf_hbm, emb_hbm, wide_hbm,
                    fm_out, ws_out,
                    idxf_v, rows0, rows1, rows2, wout_v, wsum_v,
                    sem_g, sem_w, sem_w2):
    w = lax.axis_index("s") * _NC + lax.axis_index("c")
    base = w * _CH

    pltpu.sync_copy(idxf_hbm.at[w], idxf_v)

    rows = (rows0, rows1, rows2)

    wide_cps = [pltpu.async_copy(wide_hbm.at[idxf_v.at[f]], wout_v.at[f],
                                 sem_w)
                for f in range(_F)]

    gcp = [None] * 3
    wcp = [None] * 3
    gcp[0] = pltpu.async_copy(emb_hbm.at[idxf_v.at[0]], rows[0], sem_g)
    for f in range(_F):
        b = f % 3
        if f + 1 < _F:
            nb = (f + 1) % 3
            if wcp[nb] is not None:
                wcp[nb].wait()
            gcp[nb] = pltpu.async_copy(emb_hbm.at[idxf_v.at[f + 1]],
                                       rows[nb], sem_g)
        gcp[b].wait()
        wcp[b] = pltpu.async_copy(rows[b], fm_out.at[f, pl.ds(base, _CH)],
                                  sem_w2)
    for b in range(3):
        wcp[b].wait()

    for cp in wide_cps:
        cp.wait()
    for g in range(_CH // _L):
        acc = wout_v[0, pl.ds(g * _L, _L)]
        for f in range(1, _F):
            acc = acc + wout_v[f, pl.ds(g * _L, _L)]
        wsum_v[pl.ds(g * _L, _L)] = acc
    pltpu.sync_copy(wsum_v, ws_out.at[0, pl.ds(base, _CH)])


def _sc_gather(idxf, emb, wide):
    mesh = plsc.VectorSubcoreMesh(core_axis_name="c", subcore_axis_name="s",
                                  num_cores=_NC, num_subcores=_NS)
    f32 = jnp.float32
    return pl.kernel(
        _sc_gather_body,
        out_type=(
            jax.ShapeDtypeStruct((_F, _B, _E), f32),
            jax.ShapeDtypeStruct((1, _B), f32),
        ),
        mesh=mesh,
        scratch_types=[
            pltpu.VMEM((_F, _CH), jnp.int32),
            pltpu.VMEM((_CH, _E), f32),
            pltpu.VMEM((_CH, _E), f32),
            pltpu.VMEM((_CH, _E), f32),
            pltpu.VMEM((_F, _CH), f32),
            pltpu.VMEM((_CH,), f32),
            pltpu.SemaphoreType.DMA,
            pltpu.SemaphoreType.DMA,
            pltpu.SemaphoreType.DMA,
        ],
        name="two_tower_sc_gather",
    )(idxf, emb, wide)


def _tower_from_refs(x3_ref, W1, b1, W2, b2, W3, b3):
    f32 = jnp.float32
    x3 = [x3_ref[f] for f in range(_F)]
    s = x3[0]
    ss = s * s
    for f in range(1, _F):
        e = x3[f]
        s = s + e
        ss = ss + e * e
    fm = 0.5 * (s * s - ss)
    x = jnp.concatenate(x3, axis=1)
    h = jnp.dot(x.astype(jnp.bfloat16), W1[:],
                preferred_element_type=f32) + b1[:]
    h = jnp.maximum(h, 0.0)
    h = jnp.dot(h.astype(jnp.bfloat16), W2[:],
                preferred_element_type=f32) + b2[:]
    h = jnp.maximum(h, 0.0)
    d = jnp.dot(h.astype(jnp.bfloat16), W3[:],
                preferred_element_type=f32) + b3[:]
    return fm, d


def _tc_user_body(ux_ref, uW1, ub1, uW2, ub2, uW3, ub3,
                  fm_ref, d_ref):
    fm, d = _tower_from_refs(ux_ref, uW1, ub1, uW2, ub2, uW3, ub3)
    fm_ref[:] = fm
    d_ref[:] = d


def _tc_final_body(ix_ref, iW1, ib1, iW2, ib2, iW3, ib3,
                   fmu_ref, du_ref, wsu_ref, wsi_ref, out_ref):
    fm_i, d_i = _tower_from_refs(ix_ref, iW1, ib1, iW2, ib2, iW3, ib3)
    logit = (wsu_ref[:] * wsi_ref[:]
             + jnp.sum(fmu_ref[:] * fm_i, axis=1, keepdims=True)
             + jnp.sum(du_ref[:] * d_i, axis=1, keepdims=True))
    out_ref[:] = jax.nn.sigmoid(logit)


_BS = 1024


def _xmap(i):
    return (i, 0)


def _x3map(i):
    return (0, i, 0)


def _wmap(i):
    return (0, 0)


def _tc_user(u_x, uW1, ub1, uW2, ub2, uW3, ub3):
    f32 = jnp.float32
    full = lambda a: pl.BlockSpec(a.shape, _wmap)
    return pl.pallas_call(
        _tc_user_body,
        grid=(_B // _BS,),
        in_specs=[
            pl.BlockSpec((_F, _BS, _E), _x3map),
        ] + [full(a) for a in (uW1, ub1, uW2, ub2, uW3, ub3)],
        out_specs=(
            pl.BlockSpec((_BS, _E), _xmap),
            pl.BlockSpec((_BS, _E), _xmap),
        ),
        out_shape=(
            jax.ShapeDtypeStruct((_B, _E), f32),
            jax.ShapeDtypeStruct((_B, _E), f32),
        ),
        compiler_params=pltpu.CompilerParams(
            dimension_semantics=("arbitrary",),
        ),
        name="two_tower_tc_user",
    )(u_x, uW1, ub1, uW2, ub2, uW3, ub3)


def _tc_final(i_x, iW1, ib1, iW2, ib2, iW3, ib3, fm_u, d_u, ws_u, ws_i):
    f32 = jnp.float32
    full = lambda a: pl.BlockSpec(a.shape, _wmap)
    return pl.pallas_call(
        _tc_final_body,
        grid=(_B // _BS,),
        in_specs=[
            pl.BlockSpec((_F, _BS, _E), _x3map),
        ] + [full(a) for a in (iW1, ib1, iW2, ib2, iW3, ib3)] + [
            pl.BlockSpec((_BS, _E), _xmap),
            pl.BlockSpec((_BS, _E), _xmap),
            pl.BlockSpec((_BS, 1), _xmap),
            pl.BlockSpec((_BS, 1), _xmap),
        ],
        out_specs=pl.BlockSpec((_BS, 1), _xmap),
        out_shape=jax.ShapeDtypeStruct((_B, 1), f32),
        compiler_params=pltpu.CompilerParams(
            dimension_semantics=("arbitrary",),
        ),
        name="two_tower_tc_final",
    )(i_x, iW1, ib1, iW2, ib2, iW3, ib3, fm_u, d_u, ws_u, ws_i)


def kernel(inputs, user_emb, user_wide, item_emb, item_wide,
           uW1, ub1, uW2, ub2, uW3, ub3,
           iW1, ib1, iW2, ib2, iW3, ib3):
    i32 = jnp.int32
    bf16 = jnp.bfloat16
    off = (jnp.arange(_F, dtype=i32) * _V)[None, :]
    u_flat = inputs[:, :_F].astype(i32) + off
    i_flat = inputs[:, _F:].astype(i32) + off
    u_idxf = u_flat.reshape(_NW, _CH, _F).transpose(0, 2, 1)
    i_idxf = i_flat.reshape(_NW, _CH, _F).transpose(0, 2, 1)

    u_fm, u_ws = _sc_gather(u_idxf, user_emb.reshape(_TWT, _E),
                            user_wide.reshape(_TWT))
    i_fm, i_ws = _sc_gather(i_idxf, item_emb.reshape(_TWT, _E),
                            item_wide.reshape(_TWT))

    fm_u, d_u = _tc_user(
        u_fm,
        uW1.astype(bf16), ub1.reshape(1, -1), uW2.astype(bf16),
        ub2.reshape(1, -1), uW3.astype(bf16), ub3.reshape(1, -1))

    i_fm, i_ws, fm_u = lax.optimization_barrier((i_fm, i_ws, fm_u))

    pred = _tc_final(
        i_fm,
        iW1.astype(bf16), ib1.reshape(1, -1), iW2.astype(bf16),
        ib2.reshape(1, -1), iW3.astype(bf16), ib3.reshape(1, -1),
        fm_u, d_u, u_ws.reshape(_B, 1), i_ws.reshape(_B, 1))
    return pred


# speedup vs baseline: 1.1746x; 1.0002x over previous
import jax
import jax.numpy as jnp
from jax import lax
from jax.experimental import pallas as pl
from jax.experimental.pallas import tpu as pltpu
from jax.experimental.pallas import tpu_sc as plsc

_B = 4096
_F = 13
_V = 1000
_E = 128
_NC = 2
_NS = 16
_NW = _NC * _NS
_BPW = _B // _NW
_CH = _BPW
_L = 16

_TWT = _F * _V


def _sc_gather_body(idxf_hbm, emb_hbm, wide_hbm,
                    fm_out, ws_out,
                    idxf_v, rows0, rows1, rows2, wout_v, wsum_v,
                    sem_g, sem_w, sem_w2):
    w = lax.axis_index("s") * _NC + lax.axis_index("c")
    base = w * _CH

    pltpu.sync_copy(idxf_hbm.at[w], idxf_v)

    rows = (rows0, rows1, rows2)

    wide_cps = [pltpu.async_copy(wide_hbm.at[idxf_v.at[f]], wout_v.at[f],
                                 sem_w)
                for f in range(_F)]

    gcp = [None] * 3
    wcp = [None] * 3
    gcp[0] = pltpu.async_copy(emb_hbm.at[idxf_v.at[0]], rows[0], sem_g)
    for f in range(_F):
        b = f % 3
        if f + 1 < _F:
            nb = (f + 1) % 3
            if wcp[nb] is not None:
                wcp[nb].wait()
            gcp[nb] = pltpu.async_copy(emb_hbm.at[idxf_v.at[f + 1]],
                                       rows[nb], sem_g)
        gcp[b].wait()
        wcp[b] = pltpu.async_copy(rows[b], fm_out.at[f, pl.ds(base, _CH)],
                                  sem_w2)
    for b in range(3):
        wcp[b].wait()

    for cp in wide_cps:
        cp.wait()
    for g in range(_CH // _L):
        acc = wout_v[0, pl.ds(g * _L, _L)]
        for f in range(1, _F):
            acc = acc + wout_v[f, pl.ds(g * _L, _L)]
        wsum_v[pl.ds(g * _L, _L)] = acc
    pltpu.sync_copy(wsum_v, ws_out.at[0, pl.ds(base, _CH)])


def _sc_gather(idxf, emb, wide):
    mesh = plsc.VectorSubcoreMesh(core_axis_name="c", subcore_axis_name="s",
                                  num_cores=_NC, num_subcores=_NS)
    f32 = jnp.float32
    return pl.kernel(
        _sc_gather_body,
        out_type=(
            jax.ShapeDtypeStruct((_F, _B, _E), f32),
            jax.ShapeDtypeStruct((1, _B), f32),
        ),
        mesh=mesh,
        scratch_types=[
            pltpu.VMEM((_F, _CH), jnp.int32),
            pltpu.VMEM((_CH, _E), f32),
            pltpu.VMEM((_CH, _E), f32),
            pltpu.VMEM((_CH, _E), f32),
            pltpu.VMEM((_F, _CH), f32),
            pltpu.VMEM((_CH,), f32),
            pltpu.SemaphoreType.DMA,
            pltpu.SemaphoreType.DMA,
            pltpu.SemaphoreType.DMA,
        ],
        name="two_tower_sc_gather",
    )(idxf, emb, wide)


def _tower_from_refs(x3_ref, W1, b1, W2, b2, W3, b3):
    f32 = jnp.float32
    x3 = [x3_ref[f] for f in range(_F)]
    s = x3[0]
    ss = s * s
    for f in range(1, _F):
        e = x3[f]
        s = s + e
        ss = ss + e * e
    fm = 0.5 * (s * s - ss)
    x = jnp.concatenate(x3, axis=1)
    h = jnp.dot(x.astype(jnp.bfloat16), W1[:],
                preferred_element_type=f32) + b1[:]
    h = jnp.maximum(h, 0.0)
    h = jnp.dot(h.astype(jnp.bfloat16), W2[:],
                preferred_element_type=f32) + b2[:]
    h = jnp.maximum(h, 0.0)
    d = jnp.dot(h.astype(jnp.bfloat16), W3[:],
                preferred_element_type=f32) + b3[:]
    return fm, d


def _tc_user_body(ux_ref, uW1, ub1, uW2, ub2, uW3, ub3,
                  fm_ref, d_ref):
    fm, d = _tower_from_refs(ux_ref, uW1, ub1, uW2, ub2, uW3, ub3)
    fm_ref[:] = fm
    d_ref[:] = d


def _tc_final_body(ix_ref, iW1, ib1, iW2, ib2, iW3, ib3,
                   fmu_ref, du_ref, wsu_ref, wsi_ref, out_ref):
    fm_i, d_i = _tower_from_refs(ix_ref, iW1, ib1, iW2, ib2, iW3, ib3)
    logit = (wsu_ref[:] * wsi_ref[:]
             + jnp.sum(fmu_ref[:] * fm_i, axis=1, keepdims=True)
             + jnp.sum(du_ref[:] * d_i, axis=1, keepdims=True))
    out_ref[:] = jax.nn.sigmoid(logit)


_BS = 1024


def _xmap(i):
    return (i, 0)


def _x3map(i):
    return (0, i, 0)


def _wmap(i):
    return (0, 0)


def _tc_user(u_x, uW1, ub1, uW2, ub2, uW3, ub3):
    f32 = jnp.float32
    full = lambda a: pl.BlockSpec(a.shape, _wmap)
    return pl.pallas_call(
        _tc_user_body,
        grid=(_B // _BS,),
        in_specs=[
            pl.BlockSpec((_F, _BS, _E), _x3map),
        ] + [full(a) for a in (uW1, ub1, uW2, ub2, uW3, ub3)],
        out_specs=(
            pl.BlockSpec((_BS, _E), _xmap),
            pl.BlockSpec((_BS, _E), _xmap),
        ),
        out_shape=(
            jax.ShapeDtypeStruct((_B, _E), f32),
            jax.ShapeDtypeStruct((_B, _E), f32),
        ),
        compiler_params=pltpu.CompilerParams(
            dimension_semantics=("arbitrary",),
        ),
        name="two_tower_tc_user",
    )(u_x, uW1, ub1, uW2, ub2, uW3, ub3)


def _tc_final(i_x, iW1, ib1, iW2, ib2, iW3, ib3, fm_u, d_u, ws_u, ws_i):
    f32 = jnp.float32
    full = lambda a: pl.BlockSpec(a.shape, _wmap)
    return pl.pallas_call(
        _tc_final_body,
        grid=(_B // _BS,),
        in_specs=[
            pl.BlockSpec((_F, _BS, _E), _x3map),
        ] + [full(a) for a in (iW1, ib1, iW2, ib2, iW3, ib3)] + [
            pl.BlockSpec((_BS, _E), _xmap),
            pl.BlockSpec((_BS, _E), _xmap),
            pl.BlockSpec((_BS, 1), _xmap),
            pl.BlockSpec((_BS, 1), _xmap),
        ],
        out_specs=pl.BlockSpec((_BS, 1), _xmap),
        out_shape=jax.ShapeDtypeStruct((_B, 1), f32),
        compiler_params=pltpu.CompilerParams(
            dimension_semantics=("arbitrary",),
        ),
        name="two_tower_tc_final",
    )(i_x, iW1, ib1, iW2, ib2, iW3, ib3, fm_u, d_u, ws_u, ws_i)


def kernel(inputs, user_emb, user_wide, item_emb, item_wide,
           uW1, ub1, uW2, ub2, uW3, ub3,
           iW1, ib1, iW2, ib2, iW3, ib3):
    i32 = jnp.int32
    bf16 = jnp.bfloat16
    off = (jnp.arange(_F, dtype=i32) * _V)[None, :]
    u_flat = inputs[:, :_F].astype(i32) + off
    i_flat = inputs[:, _F:].astype(i32) + off
    u_idxf = u_flat.reshape(_NW, _CH, _F).transpose(0, 2, 1)
    i_idxf = i_flat.reshape(_NW, _CH, _F).transpose(0, 2, 1)

    u_fm, u_ws = _sc_gather(u_idxf, user_emb.reshape(_TWT, _E),
                            user_wide.reshape(_TWT))
    i_fm, i_ws = _sc_gather(i_idxf, item_emb.reshape(_TWT, _E),
                            item_wide.reshape(_TWT))

    fm_u, d_u = _tc_user(
        u_fm,
        uW1.astype(bf16), ub1.reshape(1, -1), uW2.astype(bf16),
        ub2.reshape(1, -1), uW3.astype(bf16), ub3.reshape(1, -1))

    i_fm, i_ws, fm_u = lax.optimization_barrier((i_fm, i_ws, fm_u))

    pred = _tc_final(
        i_fm,
        iW1.astype(bf16), ib1.reshape(1, -1), iW2.astype(bf16),
        ib2.reshape(1, -1), iW3.astype(bf16), ib3.reshape(1, -1),
        fm_u, d_u, u_ws.reshape(_B, 1), i_ws.reshape(_B, 1))
    return pred
